# baseline (speedup 1.0000x reference)
import functools

import jax
import jax.numpy as jnp
from jax import lax
from jax.experimental import pallas as pl
from jax.experimental.pallas import tpu as pltpu
from jax.experimental.pallas import tpu_sc as plsc

B = 4096
EMB = 128
T_LEN = 50
G_LEN = 20
HID = 1024
NC = 2
NS = 16
NW = NC * NS
NCHUNK = 4
NB = B // NCHUNK
BPW = NB // NW
BB = 512


def _sc_pool_body(titleT, tagsT, ch_ids, cat_ids,
                  title_tab, tags_tab, ch_tab, cat_tab,
                  outT, outG, outC, outK,
                  idxT, idxG, idxC, idxK,
                  accT, accG, rowsC, rowsK,
                  semT, semG, semCK):
    wid = lax.axis_index("s") * NC + lax.axis_index("c")
    base = wid * BPW

    pltpu.sync_copy(titleT.at[wid], idxT)
    pltpu.sync_copy(tagsT.at[wid], idxG)
    pltpu.sync_copy(ch_ids.at[pl.ds(base, BPW)], idxC)
    pltpu.sync_copy(cat_ids.at[pl.ds(base, BPW)], idxK)

    cpC = pltpu.async_copy(ch_tab.at[idxC], rowsC, semCK)
    cpK = pltpu.async_copy(cat_tab.at[idxK], rowsK, semCK)

    zero = jnp.zeros((16,), jnp.float32)

    def zrow(j, c):
        for k in range(EMB // 16):
            accT[j, pl.ds(k * 16, 16)] = zero
            accG[j, pl.ds(k * 16, 16)] = zero
        return c

    lax.fori_loop(0, BPW, zrow, 0)

    def fireT(t, c):
        pltpu.async_copy(title_tab.at[idxT.at[t]], accT, semT, add=True)
        return c

    lax.fori_loop(0, T_LEN, fireT, 0)

    def fireG(t, c):
        pltpu.async_copy(tags_tab.at[idxG.at[t]], accG, semG, add=True)
        return c

    lax.fori_loop(0, G_LEN, fireG, 0)

    def drainT(t, c):
        pltpu.make_async_copy(title_tab.at[idxT.at[0]], accT, semT).wait()
        return c

    lax.fori_loop(0, T_LEN, drainT, 0)

    def drainG(t, c):
        pltpu.make_async_copy(tags_tab.at[idxG.at[0]], accG, semG).wait()
        return c

    lax.fori_loop(0, G_LEN, drainG, 0)
    cpC.wait()
    cpK.wait()

    pltpu.sync_copy(accT, outT.at[pl.ds(base, BPW)])
    pltpu.sync_copy(accG, outG.at[pl.ds(base, BPW)])
    pltpu.sync_copy(rowsC, outC.at[pl.ds(base, BPW)])
    pltpu.sync_copy(rowsK, outK.at[pl.ds(base, BPW)])


_sc_pool = pl.kernel(
    _sc_pool_body,
    out_type=[jax.ShapeDtypeStruct((NB, EMB), jnp.float32)] * 4,
    mesh=plsc.VectorSubcoreMesh(core_axis_name="c", subcore_axis_name="s"),
    scratch_types=[
        pltpu.VMEM((T_LEN, BPW), jnp.int32),
        pltpu.VMEM((G_LEN, BPW), jnp.int32),
        pltpu.VMEM((BPW,), jnp.int32),
        pltpu.VMEM((BPW,), jnp.int32),
        pltpu.VMEM((BPW, EMB), jnp.float32),
        pltpu.VMEM((BPW, EMB), jnp.float32),
        pltpu.VMEM((BPW, EMB), jnp.float32),
        pltpu.VMEM((BPW, EMB), jnp.float32),
        pltpu.SemaphoreType.DMA,
        pltpu.SemaphoreType.DMA,
        pltpu.SemaphoreType.DMA,
    ],
)


def _mlp_body(sumT, sumG, rowC, rowK, lenT, lenG, xnum, W1m, W1n, b1, W2t,
              b2, out):
    invT = 1.0 / jnp.maximum(lenT[...], 1.0)
    invG = 1.0 / jnp.maximum(lenG[...], 1.0)
    x = jnp.concatenate(
        [sumT[...] * invT, sumG[...] * invG, rowC[...], rowK[...]], axis=1)
    h = jnp.dot(x, W1m[...], preferred_element_type=jnp.float32)
    xn = xnum[...]
    h += xn[:, 0:1] * W1n[0:1, :] + xn[:, 1:2] * W1n[1:2, :]
    h = jnp.maximum(h + b1[...], 0.0)
    out[...] = jnp.sum(h * W2t[...], axis=1, keepdims=True) + b2[...]


_mlp = pl.pallas_call(
    _mlp_body,
    grid=(NB // BB,),
    in_specs=[
        pl.BlockSpec((BB, EMB), lambda i: (i, 0)),
        pl.BlockSpec((BB, EMB), lambda i: (i, 0)),
        pl.BlockSpec((BB, EMB), lambda i: (i, 0)),
        pl.BlockSpec((BB, EMB), lambda i: (i, 0)),
        pl.BlockSpec((BB, 1), lambda i: (i, 0)),
        pl.BlockSpec((BB, 1), lambda i: (i, 0)),
        pl.BlockSpec((BB, 2), lambda i: (i, 0)),
        pl.BlockSpec((4 * EMB, HID), lambda i: (0, 0)),
        pl.BlockSpec((2, HID), lambda i: (0, 0)),
        pl.BlockSpec((1, HID), lambda i: (0, 0)),
        pl.BlockSpec((1, HID), lambda i: (0, 0)),
        pl.BlockSpec((1, 1), lambda i: (0, 0)),
    ],
    out_specs=pl.BlockSpec((BB, 1), lambda i: (i, 0)),
    out_shape=jax.ShapeDtypeStruct((NB, 1), jnp.float32),
)


def kernel(title_ids, title_len, tags_ids, tags_len, ch_id, cat_id, x_num,
           title_table, tags_table, ch_table, cat_table, W1, b1, W2, b2):
    W1m = W1[:4 * EMB]
    W1n = W1[4 * EMB:]
    b1r = b1.reshape(1, HID)
    W2t = W2.reshape(1, HID)
    b2r = b2.reshape(1, 1)

    lenT = title_len.astype(jnp.float32).reshape(B, 1)
    lenG = tags_len.astype(jnp.float32).reshape(B, 1)

    outs = []
    for h in range(NCHUNK):
        sl = slice(h * NB, (h + 1) * NB)
        titleT = title_ids[sl].T.reshape(T_LEN, NW, BPW).transpose(1, 0, 2)
        tagsT = tags_ids[sl].T.reshape(G_LEN, NW, BPW).transpose(1, 0, 2)
        sumT, sumG, rowC, rowK = _sc_pool(
            titleT, tagsT, ch_id[sl], cat_id[sl],
            title_table, tags_table, ch_table, cat_table)
        outs.append(_mlp(sumT, sumG, rowC, rowK, lenT[sl], lenG[sl],
                         x_num[sl], W1m, W1n, b1r, W2t, b2r))
    return jnp.concatenate(outs, axis=0)[:, 0]


# speedup vs baseline: 1.1850x; 1.1850x over previous
import functools

import jax
import jax.numpy as jnp
from jax import lax
from jax.experimental import pallas as pl
from jax.experimental.pallas import tpu as pltpu
from jax.experimental.pallas import tpu_sc as plsc

B = 4096
EMB = 128
T_LEN = 50
G_LEN = 20
HID = 1024
NC = 2
NS = 16
NW = NC * NS
NCHUNK = 1
NB = B // NCHUNK
BPW = NB // NW
BB = 512


def _sc_pool_body(titleT, tagsT, ch_ids, cat_ids,
                  title_tab, tags_tab, ch_tab, cat_tab,
                  outT, outG, outC, outK,
                  idxT, idxG, idxC, idxK,
                  accT, accG, rowsC, rowsK,
                  semT, semG, semCK):
    wid = lax.axis_index("s") * NC + lax.axis_index("c")
    base = wid * BPW

    pltpu.sync_copy(titleT.at[wid], idxT)
    pltpu.sync_copy(tagsT.at[wid], idxG)
    pltpu.sync_copy(ch_ids.at[pl.ds(base, BPW)], idxC)
    pltpu.sync_copy(cat_ids.at[pl.ds(base, BPW)], idxK)

    cpC = pltpu.async_copy(ch_tab.at[idxC], rowsC, semCK)
    cpK = pltpu.async_copy(cat_tab.at[idxK], rowsK, semCK)

    zero = jnp.zeros((16,), jnp.float32)

    def zrow(j, c):
        for k in range(EMB // 16):
            accT[j, pl.ds(k * 16, 16)] = zero
            accG[j, pl.ds(k * 16, 16)] = zero
        return c

    lax.fori_loop(0, BPW, zrow, 0)

    def fireT(t, c):
        pltpu.async_copy(title_tab.at[idxT.at[t]], accT, semT, add=True)
        return c

    lax.fori_loop(0, T_LEN, fireT, 0)

    def fireG(t, c):
        pltpu.async_copy(tags_tab.at[idxG.at[t]], accG, semG, add=True)
        return c

    lax.fori_loop(0, G_LEN, fireG, 0)

    def drainT(t, c):
        pltpu.make_async_copy(title_tab.at[idxT.at[0]], accT, semT).wait()
        return c

    lax.fori_loop(0, T_LEN, drainT, 0)

    def drainG(t, c):
        pltpu.make_async_copy(tags_tab.at[idxG.at[0]], accG, semG).wait()
        return c

    lax.fori_loop(0, G_LEN, drainG, 0)
    cpC.wait()
    cpK.wait()

    pltpu.sync_copy(accT, outT.at[pl.ds(base, BPW)])
    pltpu.sync_copy(accG, outG.at[pl.ds(base, BPW)])
    pltpu.sync_copy(rowsC, outC.at[pl.ds(base, BPW)])
    pltpu.sync_copy(rowsK, outK.at[pl.ds(base, BPW)])


_sc_pool = pl.kernel(
    _sc_pool_body,
    out_type=[jax.ShapeDtypeStruct((NB, EMB), jnp.float32)] * 4,
    mesh=plsc.VectorSubcoreMesh(core_axis_name="c", subcore_axis_name="s"),
    scratch_types=[
        pltpu.VMEM((T_LEN, BPW), jnp.int32),
        pltpu.VMEM((G_LEN, BPW), jnp.int32),
        pltpu.VMEM((BPW,), jnp.int32),
        pltpu.VMEM((BPW,), jnp.int32),
        pltpu.VMEM((BPW, EMB), jnp.float32),
        pltpu.VMEM((BPW, EMB), jnp.float32),
        pltpu.VMEM((BPW, EMB), jnp.float32),
        pltpu.VMEM((BPW, EMB), jnp.float32),
        pltpu.SemaphoreType.DMA,
        pltpu.SemaphoreType.DMA,
        pltpu.SemaphoreType.DMA,
    ],
)


def _mlp_body(sumT, sumG, rowC, rowK, lenT, lenG, xnum, W1m, W1n, b1, W2t,
              b2, out):
    invT = 1.0 / jnp.maximum(lenT[...], 1.0)
    invG = 1.0 / jnp.maximum(lenG[...], 1.0)
    x = jnp.concatenate(
        [sumT[...] * invT, sumG[...] * invG, rowC[...], rowK[...]], axis=1)
    h = jnp.dot(x.astype(jnp.bfloat16), W1m[...],
                preferred_element_type=jnp.float32)
    xn = xnum[...]
    h += xn[:, 0:1] * W1n[0:1, :] + xn[:, 1:2] * W1n[1:2, :]
    h = jnp.maximum(h + b1[...], 0.0)
    out[...] = jnp.sum(h * W2t[...], axis=1, keepdims=True) + b2[...]


_mlp = pl.pallas_call(
    _mlp_body,
    grid=(NB // BB,),
    in_specs=[
        pl.BlockSpec((BB, EMB), lambda i: (i, 0)),
        pl.BlockSpec((BB, EMB), lambda i: (i, 0)),
        pl.BlockSpec((BB, EMB), lambda i: (i, 0)),
        pl.BlockSpec((BB, EMB), lambda i: (i, 0)),
        pl.BlockSpec((BB, 1), lambda i: (i, 0)),
        pl.BlockSpec((BB, 1), lambda i: (i, 0)),
        pl.BlockSpec((BB, 2), lambda i: (i, 0)),
        pl.BlockSpec((4 * EMB, HID), lambda i: (0, 0)),
        pl.BlockSpec((2, HID), lambda i: (0, 0)),
        pl.BlockSpec((1, HID), lambda i: (0, 0)),
        pl.BlockSpec((1, HID), lambda i: (0, 0)),
        pl.BlockSpec((1, 1), lambda i: (0, 0)),
    ],
    out_specs=pl.BlockSpec((BB, 1), lambda i: (i, 0)),
    out_shape=jax.ShapeDtypeStruct((NB, 1), jnp.float32),
)


def kernel(title_ids, title_len, tags_ids, tags_len, ch_id, cat_id, x_num,
           title_table, tags_table, ch_table, cat_table, W1, b1, W2, b2):
    W1m = W1[:4 * EMB].astype(jnp.bfloat16)
    W1n = W1[4 * EMB:]
    b1r = b1.reshape(1, HID)
    W2t = W2.reshape(1, HID)
    b2r = b2.reshape(1, 1)

    lenT = title_len.astype(jnp.float32).reshape(B, 1)
    lenG = tags_len.astype(jnp.float32).reshape(B, 1)

    outs = []
    for h in range(NCHUNK):
        sl = slice(h * NB, (h + 1) * NB)
        titleT = title_ids[sl].T.reshape(T_LEN, NW, BPW).transpose(1, 0, 2)
        tagsT = tags_ids[sl].T.reshape(G_LEN, NW, BPW).transpose(1, 0, 2)
        sumT, sumG, rowC, rowK = _sc_pool(
            titleT, tagsT, ch_id[sl], cat_id[sl],
            title_table, tags_table, ch_table, cat_table)
        outs.append(_mlp(sumT, sumG, rowC, rowK, lenT[sl], lenG[sl],
                         x_num[sl], W1m, W1n, b1r, W2t, b2r))
    out = outs[0] if NCHUNK == 1 else jnp.concatenate(outs, axis=0)
    return out[:, 0]


# speedup vs baseline: 1.2035x; 1.0156x over previous
import functools

import jax
import jax.numpy as jnp
from jax import lax
from jax.experimental import pallas as pl
from jax.experimental.pallas import tpu as pltpu
from jax.experimental.pallas import tpu_sc as plsc

B = 4096
EMB = 128
T_LEN = 50
G_LEN = 20
HID = 1024
NC = 2
NS = 16
NW = NC * NS
NCHUNK = 1
NB = B // NCHUNK
BPW = NB // NW
BB = 2048


def _sc_pool_body(titleT, tagsT, ch_ids, cat_ids,
                  title_tab, tags_tab, ch_tab, cat_tab,
                  outT, outG, outC, outK,
                  idxT, idxG, idxC, idxK,
                  accT, accG, rowsC, rowsK,
                  semT, semG, semCK):
    wid = lax.axis_index("s") * NC + lax.axis_index("c")
    base = wid * BPW

    pltpu.sync_copy(titleT.at[wid], idxT)
    pltpu.sync_copy(tagsT.at[wid], idxG)
    pltpu.sync_copy(ch_ids.at[pl.ds(base, BPW)], idxC)
    pltpu.sync_copy(cat_ids.at[pl.ds(base, BPW)], idxK)

    cpC = pltpu.async_copy(ch_tab.at[idxC], rowsC, semCK)
    cpK = pltpu.async_copy(cat_tab.at[idxK], rowsK, semCK)

    zero = jnp.zeros((16,), jnp.float32)

    def zrow(j, c):
        for k in range(EMB // 16):
            accT[j, pl.ds(k * 16, 16)] = zero
            accG[j, pl.ds(k * 16, 16)] = zero
        return c

    lax.fori_loop(0, BPW, zrow, 0)

    def fireT(t, c):
        pltpu.async_copy(title_tab.at[idxT.at[t]], accT, semT, add=True)
        return c

    lax.fori_loop(0, T_LEN, fireT, 0)

    def fireG(t, c):
        pltpu.async_copy(tags_tab.at[idxG.at[t]], accG, semG, add=True)
        return c

    lax.fori_loop(0, G_LEN, fireG, 0)

    def drainT(t, c):
        pltpu.make_async_copy(title_tab.at[idxT.at[0]], accT, semT).wait()
        return c

    lax.fori_loop(0, T_LEN, drainT, 0)

    def drainG(t, c):
        pltpu.make_async_copy(tags_tab.at[idxG.at[0]], accG, semG).wait()
        return c

    lax.fori_loop(0, G_LEN, drainG, 0)
    cpC.wait()
    cpK.wait()

    pltpu.sync_copy(accT, outT.at[pl.ds(base, BPW)])
    pltpu.sync_copy(accG, outG.at[pl.ds(base, BPW)])
    pltpu.sync_copy(rowsC, outC.at[pl.ds(base, BPW)])
    pltpu.sync_copy(rowsK, outK.at[pl.ds(base, BPW)])


_sc_pool = pl.kernel(
    _sc_pool_body,
    out_type=[jax.ShapeDtypeStruct((NB, EMB), jnp.float32)] * 4,
    mesh=plsc.VectorSubcoreMesh(core_axis_name="c", subcore_axis_name="s"),
    scratch_types=[
        pltpu.VMEM((T_LEN, BPW), jnp.int32),
        pltpu.VMEM((G_LEN, BPW), jnp.int32),
        pltpu.VMEM((BPW,), jnp.int32),
        pltpu.VMEM((BPW,), jnp.int32),
        pltpu.VMEM((BPW, EMB), jnp.float32),
        pltpu.VMEM((BPW, EMB), jnp.float32),
        pltpu.VMEM((BPW, EMB), jnp.float32),
        pltpu.VMEM((BPW, EMB), jnp.float32),
        pltpu.SemaphoreType.DMA,
        pltpu.SemaphoreType.DMA,
        pltpu.SemaphoreType.DMA,
    ],
)


def _mlp_body(sumT, sumG, rowC, rowK, lenT, lenG, xnum, W1m, W1n, b1, W2,
              b2, out):
    invT = 1.0 / jnp.maximum(lenT[...], 1.0)
    invG = 1.0 / jnp.maximum(lenG[...], 1.0)
    x = jnp.concatenate(
        [sumT[...] * invT, sumG[...] * invG, rowC[...], rowK[...]], axis=1)
    h = jnp.dot(x.astype(jnp.bfloat16), W1m[...],
                preferred_element_type=jnp.float32)
    xn = xnum[...]
    h += xn[:, 0:1] * W1n[0:1, :] + xn[:, 1:2] * W1n[1:2, :]
    h = jnp.maximum(h + b1[...], 0.0)
    out[...] = jnp.dot(h, W2[...], preferred_element_type=jnp.float32) \
        + b2[...]


_mlp = pl.pallas_call(
    _mlp_body,
    grid=(NB // BB,),
    in_specs=[
        pl.BlockSpec((BB, EMB), lambda i: (i, 0)),
        pl.BlockSpec((BB, EMB), lambda i: (i, 0)),
        pl.BlockSpec((BB, EMB), lambda i: (i, 0)),
        pl.BlockSpec((BB, EMB), lambda i: (i, 0)),
        pl.BlockSpec((BB, 1), lambda i: (i, 0)),
        pl.BlockSpec((BB, 1), lambda i: (i, 0)),
        pl.BlockSpec((BB, 2), lambda i: (i, 0)),
        pl.BlockSpec((4 * EMB, HID), lambda i: (0, 0)),
        pl.BlockSpec((2, HID), lambda i: (0, 0)),
        pl.BlockSpec((1, HID), lambda i: (0, 0)),
        pl.BlockSpec((HID, 1), lambda i: (0, 0)),
        pl.BlockSpec((1, 1), lambda i: (0, 0)),
    ],
    out_specs=pl.BlockSpec((BB, 1), lambda i: (i, 0)),
    out_shape=jax.ShapeDtypeStruct((NB, 1), jnp.float32),
)


def kernel(title_ids, title_len, tags_ids, tags_len, ch_id, cat_id, x_num,
           title_table, tags_table, ch_table, cat_table, W1, b1, W2, b2):
    W1m = W1[:4 * EMB].astype(jnp.bfloat16)
    W1n = W1[4 * EMB:]
    b1r = b1.reshape(1, HID)
    b2r = b2.reshape(1, 1)

    lenT = title_len.astype(jnp.float32).reshape(B, 1)
    lenG = tags_len.astype(jnp.float32).reshape(B, 1)

    outs = []
    for h in range(NCHUNK):
        sl = slice(h * NB, (h + 1) * NB)
        titleT = title_ids[sl].T.reshape(T_LEN, NW, BPW).transpose(1, 0, 2)
        tagsT = tags_ids[sl].T.reshape(G_LEN, NW, BPW).transpose(1, 0, 2)
        sumT, sumG, rowC, rowK = _sc_pool(
            titleT, tagsT, ch_id[sl], cat_id[sl],
            title_table, tags_table, ch_table, cat_table)
        outs.append(_mlp(sumT, sumG, rowC, rowK, lenT[sl], lenG[sl],
                         x_num[sl], W1m, W1n, b1r, W2, b2r))
    out = outs[0] if NCHUNK == 1 else jnp.concatenate(outs, axis=0)
    return out[:, 0]


# speedup vs baseline: 1.2523x; 1.0406x over previous
import functools

import numpy as np
import jax
import jax.numpy as jnp
from jax import lax
from jax.experimental import pallas as pl
from jax.experimental.pallas import tpu as pltpu
from jax.experimental.pallas import tpu_sc as plsc

B = 4096
EMB = 128
T_LEN = 50
G_LEN = 20
HID = 1024
NC = 2
NS = 16
NW = NC * NS
BPW = B // NW
BB = 2048

def _sc_pool_body(titleT, tagsT, ch_ids, cat_ids,
                  title_tab, tags_tab, ch_tab, cat_tab,
                  outT, outG, outC, outK,
                  idxT, idxG, idxC, idxK,
                  accT, accG, rowsC, rowsK,
                  semT, semG, semC, semK, semI, semO):
    wid = lax.axis_index("s") * NC + lax.axis_index("c")
    base = wid * BPW

    cpiC = pltpu.async_copy(ch_ids.at[pl.ds(base, BPW)], idxC, semI)
    cpiK = pltpu.async_copy(cat_ids.at[pl.ds(base, BPW)], idxK, semI)
    cpiT = pltpu.async_copy(titleT.at[wid], idxT, semI)
    cpiG = pltpu.async_copy(tagsT.at[wid], idxG, semI)

    zero = jnp.zeros((16,), jnp.float32)

    def zrow(j, c):
        for k in range(EMB // 16):
            accT[j, pl.ds(k * 16, 16)] = zero
            accG[j, pl.ds(k * 16, 16)] = zero
        return c

    lax.fori_loop(0, BPW, zrow, 0)

    cpiC.wait()
    cpC = pltpu.async_copy(ch_tab.at[idxC], rowsC, semC)
    cpiK.wait()
    cpK = pltpu.async_copy(cat_tab.at[idxK], rowsK, semK)

    cpiT.wait()

    def fireT(t, c):
        pltpu.async_copy(title_tab.at[idxT.at[t]], accT, semT, add=True)
        return c

    lax.fori_loop(0, T_LEN, fireT, 0)
    cpiG.wait()

    def fireG(t, c):
        pltpu.async_copy(tags_tab.at[idxG.at[t]], accG, semG, add=True)
        return c

    lax.fori_loop(0, G_LEN, fireG, 0)

    cpC.wait()
    oC = pltpu.async_copy(rowsC, outC.at[pl.ds(base, BPW)], semO)
    cpK.wait()
    oK = pltpu.async_copy(rowsK, outK.at[pl.ds(base, BPW)], semO)

    def drainT(t, c):
        pltpu.make_async_copy(title_tab.at[idxT.at[0]], accT, semT).wait()
        return c

    lax.fori_loop(0, T_LEN, drainT, 0)
    oT = pltpu.async_copy(accT, outT.at[pl.ds(base, BPW)], semO)

    def drainG(t, c):
        pltpu.make_async_copy(tags_tab.at[idxG.at[0]], accG, semG).wait()
        return c

    lax.fori_loop(0, G_LEN, drainG, 0)
    oG = pltpu.async_copy(accG, outG.at[pl.ds(base, BPW)], semO)
    oC.wait()
    oK.wait()
    oT.wait()
    oG.wait()


_sc_pool = pl.kernel(
    _sc_pool_body,
    out_type=[jax.ShapeDtypeStruct((B, EMB), jnp.float32)] * 4,
    mesh=plsc.VectorSubcoreMesh(core_axis_name="c", subcore_axis_name="s"),
    scratch_types=[
        pltpu.VMEM((T_LEN, BPW), jnp.int32),
        pltpu.VMEM((G_LEN, BPW), jnp.int32),
        pltpu.VMEM((BPW,), jnp.int32),
        pltpu.VMEM((BPW,), jnp.int32),
        pltpu.VMEM((BPW, EMB), jnp.float32),
        pltpu.VMEM((BPW, EMB), jnp.float32),
        pltpu.VMEM((BPW, EMB), jnp.float32),
        pltpu.VMEM((BPW, EMB), jnp.float32),
        pltpu.SemaphoreType.DMA,
        pltpu.SemaphoreType.DMA,
        pltpu.SemaphoreType.DMA,
        pltpu.SemaphoreType.DMA,
        pltpu.SemaphoreType.DMA,
        pltpu.SemaphoreType.DMA,
    ],
)


def _mlp_body(sumT, sumG, rowC, rowK, lenT, lenG, xnum, W1m, W1n, b1, W2,
              b2, out):
    invT = 1.0 / jnp.maximum(lenT[...], 1.0)
    invG = 1.0 / jnp.maximum(lenG[...], 1.0)
    x = jnp.concatenate(
        [sumT[...] * invT, sumG[...] * invG, rowC[...], rowK[...]],
        axis=1).astype(jnp.bfloat16)
    h = jnp.dot(x, W1m[...], preferred_element_type=jnp.float32)
    xn = xnum[...]
    h += xn[:, 0:1] * W1n[0:1, :] + xn[:, 1:2] * W1n[1:2, :]
    h = jnp.maximum(h + b1[...], 0.0)
    out[...] = (jnp.dot(h, W2[...], preferred_element_type=jnp.float32)
                + b2[...])[:, 0]


_mlp = pl.pallas_call(
    _mlp_body,
    grid=(B // BB,),
    in_specs=[
        pl.BlockSpec((BB, EMB), lambda i: (i, 0)),
        pl.BlockSpec((BB, EMB), lambda i: (i, 0)),
        pl.BlockSpec((BB, EMB), lambda i: (i, 0)),
        pl.BlockSpec((BB, EMB), lambda i: (i, 0)),
        pl.BlockSpec((BB, 1), lambda i: (i, 0)),
        pl.BlockSpec((BB, 1), lambda i: (i, 0)),
        pl.BlockSpec((BB, 2), lambda i: (i, 0)),
        pl.BlockSpec((4 * EMB, HID), lambda i: (0, 0)),
        pl.BlockSpec((2, HID), lambda i: (0, 0)),
        pl.BlockSpec((1, HID), lambda i: (0, 0)),
        pl.BlockSpec((HID, 1), lambda i: (0, 0)),
        pl.BlockSpec((1, 1), lambda i: (0, 0)),
    ],
    out_specs=pl.BlockSpec((BB,), lambda i: (i,)),
    out_shape=jax.ShapeDtypeStruct((B,), jnp.float32),
)


def kernel(title_ids, title_len, tags_ids, tags_len, ch_id, cat_id, x_num,
           title_table, tags_table, ch_table, cat_table, W1, b1, W2, b2):
    titleT = title_ids.T.reshape(T_LEN, NW, BPW).transpose(1, 0, 2)
    tagsT = tags_ids.T.reshape(G_LEN, NW, BPW).transpose(1, 0, 2)

    sumT, sumG, rowC, rowK = _sc_pool(
        titleT, tagsT, ch_id, cat_id,
        title_table, tags_table, ch_table, cat_table)

    lenT = title_len.astype(jnp.float32).reshape(B, 1)
    lenG = tags_len.astype(jnp.float32).reshape(B, 1)
    W1m = W1[:4 * EMB].astype(jnp.bfloat16)
    W1n = W1[4 * EMB:]
    b1r = b1.reshape(1, HID)
    b2r = b2.reshape(1, 1)
    return _mlp(sumT, sumG, rowC, rowK, lenT, lenG, x_num,
                W1m, W1n, b1r, W2, b2r)
